# Initial kernel scaffold; baseline (speedup 1.0000x reference)
import functools

import jax
import jax.numpy as jnp
from jax import lax
from jax.experimental import pallas as pl
from jax.experimental.pallas import tpu as pltpu
from jax.experimental.pallas import tpu_sc as plsc

_N = 10000
_D = 256
_DH = 128
_E = 160000
_NC = 2
_NS = 16
_NPAD = 10240
_RPT = _NPAD // _NS
_EPAD = 163840
_EPT = _EPAD // _NS
_W = 128
_NWIN = _EPT // _W
_RB = 1024
_NRB = _NPAD // _RB

_sc_mesh = plsc.VectorSubcoreMesh(core_axis_name="c", subcore_axis_name="s")


@functools.partial(
    pl.kernel,
    out_type=(
        jax.ShapeDtypeStruct((_NC * _NPAD, _DH), jnp.float32),
        jax.ShapeDtypeStruct((_NPAD,), jnp.float32),
    ),
    mesh=_sc_mesh,
    scratch_types=(
        pltpu.VMEM((_W,), jnp.int32),
        pltpu.VMEM((_W,), jnp.int32),
        pltpu.VMEM((_W, _DH), jnp.float32),
        pltpu.VMEM((_W,), jnp.float32),
        pltpu.VMEM_SHARED((_NPAD, _DH), jnp.float32),
        pltpu.VMEM_SHARED((_NPAD,), jnp.float32),
        pltpu.SemaphoreType.DMA,
    ),
)
def _sc_agg(src2_hbm, dst_hbm, xcat_hbm, agg_hbm, cnt_hbm,
            src_v, dst_v, rows_v, ones_v, acc_sh, cnt_sh, sem):
    c = lax.axis_index("c")
    s = lax.axis_index("s")

    def _fill_zero_row(i, carry):
        for j in range(_DH // 16):
            rows_v[i, pl.ds(j * 16, 16)] = jnp.zeros((16,), jnp.float32)
        return carry

    lax.fori_loop(0, _W, _fill_zero_row, 0)
    for j in range(_W // 16):
        ones_v[pl.ds(j * 16, 16)] = jnp.ones((16,), jnp.float32)

    def _zero_blk(k, carry):
        r0 = s * _RPT + k * _W
        pltpu.sync_copy(rows_v, acc_sh.at[pl.ds(r0, _W)])
        pltpu.sync_copy(rows_v.at[0], cnt_sh.at[pl.ds(r0, _W)])
        return carry

    lax.fori_loop(0, _RPT // _W, _zero_blk, 0)
    plsc.subcore_barrier()

    ebase = s * _EPT

    def _win(w, carry):
        base = ebase + w * _W
        pltpu.sync_copy(src2_hbm.at[pl.ds(c * _EPAD + base, _W)], src_v)
        pltpu.sync_copy(dst_hbm.at[pl.ds(base, _W)], dst_v)
        pltpu.async_copy(xcat_hbm.at[src_v], rows_v, sem).wait()
        pltpu.sync_copy(rows_v, acc_sh.at[dst_v], add=True)

        @pl.when(c == 0)
        def _():
            pltpu.sync_copy(ones_v, cnt_sh.at[dst_v], add=True)

        return carry

    lax.fori_loop(0, _NWIN, _win, 0)
    plsc.subcore_barrier()

    def _drain(k, carry):
        r0 = s * _RPT + k * _W
        pltpu.sync_copy(acc_sh.at[pl.ds(r0, _W)],
                        agg_hbm.at[pl.ds(c * _NPAD + r0, _W)])
        return carry

    lax.fori_loop(0, _RPT // _W, _drain, 0)

    @pl.when(c == 0)
    def _():
        def _drain_cnt(k, carry):
            r0 = s * _RPT + k * _W
            pltpu.sync_copy(cnt_sh.at[pl.ds(r0, _W)], cnt_hbm.at[pl.ds(r0, _W)])
            return carry

        lax.fori_loop(0, _RPT // _W, _drain_cnt, 0)


def _dense_common(agg_lo_ref, agg_hi_ref, x_lo_ref, x_hi_ref, cnt_ref,
                  wl_ref, wr_ref, b_ref):
    inv = 1.0 / jnp.maximum(cnt_ref[...], 1.0)
    m = jnp.concatenate([agg_lo_ref[...] * inv, agg_hi_ref[...] * inv], axis=1)
    xx = jnp.concatenate([x_lo_ref[...], x_hi_ref[...]], axis=1)
    h = (jnp.dot(m, wl_ref[...], preferred_element_type=jnp.float32,
                 precision=lax.Precision.HIGHEST)
         + jnp.dot(xx, wr_ref[...], preferred_element_type=jnp.float32,
                   precision=lax.Precision.HIGHEST)
         + b_ref[...])
    return h


def _dense1_body(agg_lo_ref, agg_hi_ref, x_lo_ref, x_hi_ref, cnt_ref,
                 wl_ref, wr_ref, b_ref, h_lo_ref, h_hi_ref):
    h = _dense_common(agg_lo_ref, agg_hi_ref, x_lo_ref, x_hi_ref, cnt_ref,
                      wl_ref, wr_ref, b_ref)
    h = jnp.maximum(h, 0.0)
    h_lo_ref[...] = h[:, :_DH]
    h_hi_ref[...] = h[:, _DH:]


def _dense2_body(agg_lo_ref, agg_hi_ref, x_lo_ref, x_hi_ref, cnt_ref,
                 wl_ref, wr_ref, b_ref, out_ref):
    h = _dense_common(agg_lo_ref, agg_hi_ref, x_lo_ref, x_hi_ref, cnt_ref,
                      wl_ref, wr_ref, b_ref)
    hmax = jnp.max(h, axis=1, keepdims=True)
    e = jnp.exp(h - hmax)
    lse = jnp.log(jnp.sum(e, axis=1, keepdims=True))
    out_ref[...] = h - hmax - lse


def _dense_in_specs():
    return [
        pl.BlockSpec((_RB, _DH), lambda i: (i, 0)),
        pl.BlockSpec((_RB, _DH), lambda i: (i + _NRB, 0)),
        pl.BlockSpec((_RB, _DH), lambda i: (i, 0)),
        pl.BlockSpec((_RB, _DH), lambda i: (i + _NRB, 0)),
        pl.BlockSpec((_RB, 1), lambda i: (i, 0)),
        pl.BlockSpec((_D, _D), lambda i: (0, 0)),
        pl.BlockSpec((_D, _D), lambda i: (0, 0)),
        pl.BlockSpec((1, _D), lambda i: (0, 0)),
    ]


_dense1 = pl.pallas_call(
    _dense1_body,
    grid=(_NRB,),
    in_specs=_dense_in_specs(),
    out_specs=[
        pl.BlockSpec((_RB, _DH), lambda i: (i, 0)),
        pl.BlockSpec((_RB, _DH), lambda i: (i, 0)),
    ],
    out_shape=[
        jax.ShapeDtypeStruct((_NPAD, _DH), jnp.float32),
        jax.ShapeDtypeStruct((_NPAD, _DH), jnp.float32),
    ],
)

_dense2 = pl.pallas_call(
    _dense2_body,
    grid=(_NRB,),
    in_specs=_dense_in_specs(),
    out_specs=pl.BlockSpec((_RB, _D), lambda i: (i, 0)),
    out_shape=jax.ShapeDtypeStruct((_NPAD, _D), jnp.float32),
)


def kernel(x, edge_index, W1_l, W1_r, b1, W2_l, W2_r, b2):
    ei = edge_index.astype(jnp.int32)
    src, dst = ei[0], ei[1]

    npad_e = _EPAD - _E
    pad_ar = jnp.arange(npad_e, dtype=jnp.int32)
    pad_src = (pad_ar * 577) % _N
    pad_dst = _N + pad_ar % (_NPAD - _N)
    srcp = jnp.concatenate([src, pad_src])
    dstp = jnp.concatenate([dst, pad_dst])
    src2 = jnp.concatenate([srcp, srcp + _NPAD])

    def xsplit(v):
        zpad = ((0, _NPAD - _N), (0, 0))
        return jnp.concatenate([jnp.pad(v[:, :_DH], zpad),
                                jnp.pad(v[:, _DH:], zpad)], axis=0)

    xcat = xsplit(x)
    agg1, cnt = _sc_agg(src2, dstp, xcat)
    cnt2 = cnt.reshape(_NPAD, 1)
    h_lo, h_hi = _dense1(agg1, xcat, cnt2, W1_l.T, W1_r.T, b1.reshape(1, _D))
    hcat = jnp.concatenate([h_lo, h_hi], axis=0)
    agg2, _ = _sc_agg(src2, dstp, hcat)
    out = _dense2(agg2, hcat, cnt2, W2_l.T, W2_r.T, b2.reshape(1, _D))
    return out[:_N]


# speedup vs baseline: 4.3277x; 4.3277x over previous
import functools

import jax
import jax.numpy as jnp
from jax import lax
from jax.experimental import pallas as pl
from jax.experimental.pallas import tpu as pltpu
from jax.experimental.pallas import tpu_sc as plsc

_N = 10000
_D = 256
_DH = 128
_E = 160000
_NC = 2
_NS = 16
_NPAD = 10240
_RPT = _NPAD // _NS
_EPAD = 163840
_EPT = _EPAD // _NS
_W = 128
_NWIN = _EPT // _W
_RB = 1024
_NRB = _NPAD // _RB

_sc_mesh = plsc.VectorSubcoreMesh(core_axis_name="c", subcore_axis_name="s")


@functools.partial(
    pl.kernel,
    out_type=(
        jax.ShapeDtypeStruct((_NC * _NPAD, _DH), jnp.float32),
        jax.ShapeDtypeStruct((_NPAD,), jnp.float32),
    ),
    mesh=_sc_mesh,
    scratch_types=(
        pltpu.VMEM((_W,), jnp.int32),
        pltpu.VMEM((_W,), jnp.int32),
        pltpu.VMEM((_W, _DH), jnp.float32),
        pltpu.VMEM((_W,), jnp.float32),
        pltpu.VMEM_SHARED((_NPAD, _DH), jnp.float32),
        pltpu.VMEM_SHARED((_NPAD,), jnp.float32),
        pltpu.SemaphoreType.DMA,
    ),
)
def _sc_agg(src2_hbm, dst_hbm, xcat_hbm, agg_hbm, cnt_hbm,
            src_v, dst_v, rows_v, ones_v, acc_sh, cnt_sh, sem):
    c = lax.axis_index("c")
    s = lax.axis_index("s")

    def _fill_zero_row(i, carry):
        for j in range(_DH // 16):
            rows_v[i, pl.ds(j * 16, 16)] = jnp.zeros((16,), jnp.float32)
        return carry

    lax.fori_loop(0, _W, _fill_zero_row, 0)
    for j in range(_W // 16):
        ones_v[pl.ds(j * 16, 16)] = jnp.ones((16,), jnp.float32)

    def _zero_blk(k, carry):
        r0 = s * _RPT + k * _W
        pltpu.sync_copy(rows_v, acc_sh.at[pl.ds(r0, _W)])
        pltpu.sync_copy(rows_v.at[0], cnt_sh.at[pl.ds(r0, _W)])
        return carry

    lax.fori_loop(0, _RPT // _W, _zero_blk, 0)
    plsc.subcore_barrier()

    ebase = s * _EPT

    def _win(w, carry):
        base = ebase + w * _W
        pltpu.sync_copy(src2_hbm.at[pl.ds(c * _EPAD + base, _W)], src_v)
        pltpu.sync_copy(dst_hbm.at[pl.ds(base, _W)], dst_v)
        pltpu.async_copy(xcat_hbm.at[src_v], rows_v, sem).wait()
        pltpu.sync_copy(rows_v, acc_sh.at[dst_v], add=True)

        @pl.when(c == 0)
        def _():
            pltpu.sync_copy(ones_v, cnt_sh.at[dst_v], add=True)

        return carry

    lax.fori_loop(0, _NWIN, _win, 0)
    plsc.subcore_barrier()

    def _drain(k, carry):
        r0 = s * _RPT + k * _W
        pltpu.sync_copy(acc_sh.at[pl.ds(r0, _W)],
                        agg_hbm.at[pl.ds(c * _NPAD + r0, _W)])
        return carry

    lax.fori_loop(0, _RPT // _W, _drain, 0)

    @pl.when(c == 0)
    def _():
        def _drain_cnt(k, carry):
            r0 = s * _RPT + k * _W
            pltpu.sync_copy(cnt_sh.at[pl.ds(r0, _W)], cnt_hbm.at[pl.ds(r0, _W)])
            return carry

        lax.fori_loop(0, _RPT // _W, _drain_cnt, 0)


def _dense_common(agg_lo_ref, agg_hi_ref, x_lo_ref, x_hi_ref, cnt_ref,
                  wl_ref, wr_ref, b_ref):
    inv = 1.0 / jnp.maximum(cnt_ref[...], 1.0)
    m = jnp.concatenate([agg_lo_ref[...] * inv, agg_hi_ref[...] * inv], axis=1)
    xx = jnp.concatenate([x_lo_ref[...], x_hi_ref[...]], axis=1)
    h = (jnp.dot(m, wl_ref[...], preferred_element_type=jnp.float32,
                 precision=lax.Precision.HIGHEST)
         + jnp.dot(xx, wr_ref[...], preferred_element_type=jnp.float32,
                   precision=lax.Precision.HIGHEST)
         + b_ref[...])
    return h


def _dense1_body(agg_lo_ref, agg_hi_ref, x_lo_ref, x_hi_ref, cnt_ref,
                 wl_ref, wr_ref, b_ref, h_lo_ref, h_hi_ref):
    h = _dense_common(agg_lo_ref, agg_hi_ref, x_lo_ref, x_hi_ref, cnt_ref,
                      wl_ref, wr_ref, b_ref)
    h = jnp.maximum(h, 0.0)
    h_lo_ref[...] = h[:, :_DH]
    h_hi_ref[...] = h[:, _DH:]


def _dense2_body(agg_lo_ref, agg_hi_ref, x_lo_ref, x_hi_ref, cnt_ref,
                 wl_ref, wr_ref, b_ref, out_ref):
    h = _dense_common(agg_lo_ref, agg_hi_ref, x_lo_ref, x_hi_ref, cnt_ref,
                      wl_ref, wr_ref, b_ref)
    hmax = jnp.max(h, axis=1, keepdims=True)
    e = jnp.exp(h - hmax)
    lse = jnp.log(jnp.sum(e, axis=1, keepdims=True))
    out_ref[...] = h - hmax - lse


def _dense_in_specs():
    return [
        pl.BlockSpec((_RB, _DH), lambda i: (i, 0)),
        pl.BlockSpec((_RB, _DH), lambda i: (i + _NRB, 0)),
        pl.BlockSpec((_RB, _DH), lambda i: (i, 0)),
        pl.BlockSpec((_RB, _DH), lambda i: (i + _NRB, 0)),
        pl.BlockSpec((_RB, 1), lambda i: (i, 0)),
        pl.BlockSpec((_D, _D), lambda i: (0, 0)),
        pl.BlockSpec((_D, _D), lambda i: (0, 0)),
        pl.BlockSpec((1, _D), lambda i: (0, 0)),
    ]


_dense1 = pl.pallas_call(
    _dense1_body,
    grid=(_NRB,),
    in_specs=_dense_in_specs(),
    out_specs=[
        pl.BlockSpec((_RB, _DH), lambda i: (i, 0)),
        pl.BlockSpec((_RB, _DH), lambda i: (i, 0)),
    ],
    out_shape=[
        jax.ShapeDtypeStruct((_NPAD, _DH), jnp.float32),
        jax.ShapeDtypeStruct((_NPAD, _DH), jnp.float32),
    ],
)

_dense2 = pl.pallas_call(
    _dense2_body,
    grid=(_NRB,),
    in_specs=_dense_in_specs(),
    out_specs=pl.BlockSpec((_RB, _D), lambda i: (i, 0)),
    out_shape=jax.ShapeDtypeStruct((_NPAD, _D), jnp.float32),
)


def kernel(x, edge_index, W1_l, W1_r, b1, W2_l, W2_r, b2):
    ei = edge_index.astype(jnp.int32)
    src, dst = ei[0], ei[1]

    npad_e = _EPAD - _E
    pad_ar = jnp.arange(npad_e, dtype=jnp.int32)
    pad_src = (pad_ar * 577) % _N
    pad_dst = _N + pad_ar % (_NPAD - _N)
    srcp = jnp.concatenate([src, pad_src])
    dstp = jnp.concatenate([dst, pad_dst])
    src2 = jnp.concatenate([srcp, srcp + _NPAD])

    def xsplit(v):
        zpad = ((0, _NPAD - _N), (0, 0))
        return jnp.concatenate([jnp.pad(v[:, :_DH], zpad),
                                jnp.pad(v[:, _DH:], zpad)], axis=0)

    xcat = xsplit(x)
    agg1, cnt = _sc_agg(src2, dstp, xcat)
    cnt2 = cnt.reshape(_NPAD, 1)
    h_lo, h_hi = _dense1(agg1, agg1, xcat, xcat, cnt2,
                         W1_l.T, W1_r.T, b1.reshape(1, _D))
    hcat = jnp.concatenate([h_lo, h_hi], axis=0)
    agg2, _ = _sc_agg(src2, dstp, hcat)
    out = _dense2(agg2, agg2, hcat, hcat, cnt2,
                  W2_l.T, W2_r.T, b2.reshape(1, _D))
    return out[:_N]


# speedup vs baseline: 8.2843x; 1.9142x over previous
import functools

import jax
import jax.numpy as jnp
from jax import lax
from jax.experimental import pallas as pl
from jax.experimental.pallas import tpu as pltpu
from jax.experimental.pallas import tpu_sc as plsc

_N = 10000
_D = 256
_DH = 128
_E = 160000
_NC = 2
_NS = 16
_NPAD = 10240
_RPT = _NPAD // _NS
_EPAD = 163840
_EPT = _EPAD // _NS
_W = 128
_NWIN = _EPT // _W
_NWC = _NWIN // _NC
_RB = 1024
_NRB = _NPAD // _RB

_sc_mesh = plsc.VectorSubcoreMesh(core_axis_name="c", subcore_axis_name="s")


def _make_sc_agg(with_cnt):
    out_type = [jax.ShapeDtypeStruct((_NC * _NPAD, _DH), jnp.float32)]
    if with_cnt:
        out_type.append(jax.ShapeDtypeStruct((_NC, _NPAD), jnp.float32))

    scratch = (
        pltpu.VMEM((4, 2, _W), jnp.int32),
        pltpu.VMEM((2, _W, _DH), jnp.float32),
        pltpu.VMEM((_W,), jnp.float32),
        pltpu.VMEM_SHARED((_NPAD, _DH), jnp.float32),
        pltpu.VMEM_SHARED((_NPAD,), jnp.float32),
        pltpu.SemaphoreType.DMA,
        pltpu.SemaphoreType.DMA,
        pltpu.SemaphoreType.DMA,
        pltpu.SemaphoreType.DMA,
    )

    def body(edpk_hbm, xcat_hbm, agg_hbm, *rest):
        if with_cnt:
            (cnt_hbm, ed_v, rows_v, ones_v, acc_sh, cnt_sh,
             isem, gsem, ssem, csem) = rest
        else:
            (ed_v, rows_v, ones_v, acc_sh, cnt_sh,
             isem, gsem, ssem, csem) = rest
        c = lax.axis_index("c")
        s = lax.axis_index("s")

        def _issue_idx(w, slot):
            pltpu.async_copy(edpk_hbm.at[c, s, w], ed_v.at[slot], isem)

        def _wait_one(sem, dst):
            pltpu.make_async_copy(xcat_hbm.at[0], dst, sem).wait()

        def _wait_idx():
            pltpu.make_async_copy(edpk_hbm.at[0, 0, 0], ed_v.at[0],
                                  isem).wait()

        def _wait_rows(sem, b):
            pltpu.make_async_copy(xcat_hbm.at[pl.ds(0, _W)],
                                  rows_v.at[b], sem).wait()

        for k in range(4):
            _issue_idx(k, k)

        def _fill_zero_row(i, carry):
            for j in range(_DH // 16):
                rows_v[0, i, pl.ds(j * 16, 16)] = jnp.zeros((16,), jnp.float32)
            return carry

        lax.fori_loop(0, _W, _fill_zero_row, 0)

        def _zero_blk(k, carry):
            r0 = s * _RPT + k * _W
            pltpu.sync_copy(rows_v.at[0], acc_sh.at[pl.ds(r0, _W)])
            if with_cnt:
                pltpu.sync_copy(rows_v.at[0, 0], cnt_sh.at[pl.ds(r0, _W)])
            return carry

        lax.fori_loop(0, _RPT // _W, _zero_blk, 0)
        if with_cnt:
            for j in range(_W // 16):
                ones_v[pl.ds(j * 16, 16)] = jnp.ones((16,), jnp.float32)

        for b in range(2):
            _wait_idx()
            pltpu.async_copy(xcat_hbm.at[ed_v.at[b, 0]], rows_v.at[b], gsem)
        plsc.subcore_barrier()

        def _step(w, b, issue_next):
            slot = w % 4
            _wait_rows(gsem, b)
            pltpu.async_copy(rows_v.at[b], acc_sh.at[ed_v.at[slot, 1]],
                             ssem, add=True)
            in_rng = (w >= c * _NWC) & (w < (c + 1) * _NWC)
            if with_cnt:
                @pl.when(in_rng)
                def _():
                    pltpu.async_copy(ones_v, cnt_sh.at[ed_v.at[slot, 1]],
                                     csem, add=True)
            if issue_next:
                _wait_rows(ssem, b)
                if with_cnt:
                    @pl.when(in_rng)
                    def _():
                        _wait_one(csem, ones_v)
                @pl.when(w + 4 < _NWIN)
                def _():
                    _issue_idx(w + 4, slot)
                _wait_idx()
                pltpu.async_copy(xcat_hbm.at[ed_v.at[(w + 2) % 4, 0]],
                                 rows_v.at[b], gsem)

        def _pair(i, carry):
            for b in range(2):
                _step(2 * i + b, b, True)
            return carry

        lax.fori_loop(0, _NWIN // 2 - 1, _pair, 0)
        for b in range(2):
            _step(_NWIN - 2 + b, b, False)
        for b in range(2):
            _wait_rows(ssem, b)
        if with_cnt:
            @pl.when(c == 1)
            def _():
                _wait_one(csem, ones_v)
                _wait_one(csem, ones_v)

        plsc.subcore_barrier()
        r0 = s * _RPT
        pltpu.sync_copy(acc_sh.at[pl.ds(r0, _RPT)],
                        agg_hbm.at[pl.ds(c * _NPAD + r0, _RPT)])
        if with_cnt:
            pltpu.sync_copy(cnt_sh.at[pl.ds(r0, _RPT)],
                            cnt_hbm.at[c, pl.ds(r0, _RPT)])

    return pl.kernel(body, out_type=tuple(out_type), mesh=_sc_mesh,
                     scratch_types=scratch)


_sc_agg_cnt = _make_sc_agg(True)
_sc_agg = _make_sc_agg(False)


def _dense_common(agg_lo_ref, agg_hi_ref, x_lo_ref, x_hi_ref, cnt_ref,
                  wl_ref, wr_ref, b_ref):
    cnt = cnt_ref[0] + cnt_ref[1]
    inv = 1.0 / jnp.maximum(cnt, 1.0)
    m = jnp.concatenate([agg_lo_ref[...] * inv, agg_hi_ref[...] * inv], axis=1)
    xx = jnp.concatenate([x_lo_ref[...], x_hi_ref[...]], axis=1)
    h = (jnp.dot(m, wl_ref[...], preferred_element_type=jnp.float32,
                 precision=lax.Precision.HIGHEST)
         + jnp.dot(xx, wr_ref[...], preferred_element_type=jnp.float32,
                   precision=lax.Precision.HIGHEST)
         + b_ref[...])
    return h


def _dense1_body(agg_lo_ref, agg_hi_ref, x_lo_ref, x_hi_ref, cnt_ref,
                 wl_ref, wr_ref, b_ref, h_lo_ref, h_hi_ref):
    h = _dense_common(agg_lo_ref, agg_hi_ref, x_lo_ref, x_hi_ref, cnt_ref,
                      wl_ref, wr_ref, b_ref)
    h = jnp.maximum(h, 0.0)
    h_lo_ref[...] = h[:, :_DH]
    h_hi_ref[...] = h[:, _DH:]


def _dense2_body(agg_lo_ref, agg_hi_ref, x_lo_ref, x_hi_ref, cnt_ref,
                 wl_ref, wr_ref, b_ref, out_ref):
    h = _dense_common(agg_lo_ref, agg_hi_ref, x_lo_ref, x_hi_ref, cnt_ref,
                      wl_ref, wr_ref, b_ref)
    hmax = jnp.max(h, axis=1, keepdims=True)
    e = jnp.exp(h - hmax)
    lse = jnp.log(jnp.sum(e, axis=1, keepdims=True))
    out_ref[...] = h - hmax - lse


def _dense_in_specs():
    return [
        pl.BlockSpec((_RB, _DH), lambda i: (i, 0)),
        pl.BlockSpec((_RB, _DH), lambda i: (i + _NRB, 0)),
        pl.BlockSpec((_RB, _DH), lambda i: (i, 0)),
        pl.BlockSpec((_RB, _DH), lambda i: (i + _NRB, 0)),
        pl.BlockSpec((_NC, _RB, 1), lambda i: (0, i, 0)),
        pl.BlockSpec((_D, _D), lambda i: (0, 0)),
        pl.BlockSpec((_D, _D), lambda i: (0, 0)),
        pl.BlockSpec((1, _D), lambda i: (0, 0)),
    ]


_dense1 = pl.pallas_call(
    _dense1_body,
    grid=(_NRB,),
    in_specs=_dense_in_specs(),
    out_specs=[
        pl.BlockSpec((_RB, _DH), lambda i: (i, 0)),
        pl.BlockSpec((_RB, _DH), lambda i: (i, 0)),
    ],
    out_shape=[
        jax.ShapeDtypeStruct((_NPAD, _DH), jnp.float32),
        jax.ShapeDtypeStruct((_NPAD, _DH), jnp.float32),
    ],
)

_dense2 = pl.pallas_call(
    _dense2_body,
    grid=(_NRB,),
    in_specs=_dense_in_specs(),
    out_specs=pl.BlockSpec((_RB, _D), lambda i: (i, 0)),
    out_shape=jax.ShapeDtypeStruct((_NPAD, _D), jnp.float32),
)


def kernel(x, edge_index, W1_l, W1_r, b1, W2_l, W2_r, b2):
    ei = edge_index.astype(jnp.int32)
    src, dst = ei[0], ei[1]

    npad_e = _EPAD - _E
    pad_ar = jnp.arange(npad_e, dtype=jnp.int32)
    pad_src = (pad_ar * 577) % _N
    pad_dst = _N + pad_ar % (_NPAD - _N)
    srcp = jnp.concatenate([src, pad_src]).reshape(_NS, _NWIN, _W)
    dstp = jnp.concatenate([dst, pad_dst]).reshape(_NS, _NWIN, _W)
    edpk = jnp.stack([jnp.stack([srcp, dstp], axis=2),
                      jnp.stack([srcp + _NPAD, dstp], axis=2)], axis=0)

    def xsplit(v):
        zpad = ((0, _NPAD - _N), (0, 0))
        return jnp.concatenate([jnp.pad(v[:, :_DH], zpad),
                                jnp.pad(v[:, _DH:], zpad)], axis=0)

    xcat = xsplit(x)
    agg1, cnt = _sc_agg_cnt(edpk, xcat)
    cnt2 = cnt.reshape(_NC, _NPAD, 1)
    h_lo, h_hi = _dense1(agg1, agg1, xcat, xcat, cnt2,
                         W1_l.T, W1_r.T, b1.reshape(1, _D))
    hcat = jnp.concatenate([h_lo, h_hi], axis=0)
    agg2 = _sc_agg(edpk, hcat)[0]
    out = _dense2(agg2, agg2, hcat, hcat, cnt2,
                  W2_l.T, W2_r.T, b2.reshape(1, _D))
    return out[:_N]


# speedup vs baseline: 8.9649x; 1.0822x over previous
import jax
import jax.numpy as jnp
from jax import lax
from jax.experimental import pallas as pl
from jax.experimental.pallas import tpu as pltpu
from jax.experimental.pallas import tpu_sc as plsc

_N = 10000
_D = 256
_DH = 128
_E = 160000
_NC = 2
_NS = 16
_NPAD = 10240
_RPT = _NPAD // _NS
_EPAD = 163840
_EPT = _EPAD // _NS
_W = 128
_NWIN = _EPT // _W
_NWC = _NWIN // _NC
_RB = 1024
_NRB = _NPAD // _RB

_sc_mesh = plsc.VectorSubcoreMesh(core_axis_name="c", subcore_axis_name="s")


def _make_sc_agg(from_x):
    out_type = [jax.ShapeDtypeStruct((_NC * _NPAD, _DH), jnp.float32)]
    if from_x:
        out_type.append(jax.ShapeDtypeStruct((_NC, _NPAD), jnp.float32))

    scratch = (
        pltpu.VMEM((4, 2, _W), jnp.int32),
        pltpu.VMEM((2, _W, _DH), jnp.float32),
        pltpu.VMEM((_W,), jnp.float32),
        pltpu.VMEM_SHARED((_NPAD, _DH), jnp.float32),
        pltpu.VMEM_SHARED((_NPAD,), jnp.float32),
        pltpu.SemaphoreType.DMA,
        pltpu.SemaphoreType.DMA,
        pltpu.SemaphoreType.DMA,
        pltpu.SemaphoreType.DMA,
    )

    def body(edpk_hbm, x_hbm, agg_hbm, *rest):
        if from_x:
            (cnt_hbm, ed_v, rows_v, ones_v, acc_sh, cnt_sh,
             isem, gsem, ssem, csem) = rest
        else:
            (ed_v, rows_v, ones_v, acc_sh, cnt_sh,
             isem, gsem, ssem, csem) = rest
        c = lax.axis_index("c")
        s = lax.axis_index("s")

        def _gather_src(idx_ref):
            if from_x:
                return x_hbm.at[idx_ref, pl.ds(c * _DH, _DH)]
            return x_hbm.at[idx_ref]

        def _dummy_rows_src():
            if from_x:
                return x_hbm.at[pl.ds(0, _W), pl.ds(0, _DH)]
            return x_hbm.at[pl.ds(0, _W)]

        def _dummy_row_src():
            if from_x:
                return x_hbm.at[0, pl.ds(0, _DH)]
            return x_hbm.at[0]

        def _issue_idx(w, slot):
            pltpu.async_copy(edpk_hbm.at[c, s, w], ed_v.at[slot], isem)

        def _wait_cnt():
            pltpu.make_async_copy(_dummy_row_src(), ones_v, csem).wait()

        def _wait_idx():
            pltpu.make_async_copy(edpk_hbm.at[0, 0, 0], ed_v.at[0],
                                  isem).wait()

        def _wait_rows(sem, b):
            pltpu.make_async_copy(_dummy_rows_src(), rows_v.at[b], sem).wait()

        for k in range(4):
            _issue_idx(k, k)

        def _fill_zero_row(i, carry):
            for j in range(_DH // 16):
                rows_v[0, i, pl.ds(j * 16, 16)] = jnp.zeros((16,), jnp.float32)
            return carry

        lax.fori_loop(0, _W, _fill_zero_row, 0)

        def _zero_blk(k, carry):
            r0 = s * _RPT + k * _W
            pltpu.sync_copy(rows_v.at[0], acc_sh.at[pl.ds(r0, _W)])
            if from_x:
                pltpu.sync_copy(rows_v.at[0, 0], cnt_sh.at[pl.ds(r0, _W)])
            return carry

        lax.fori_loop(0, _RPT // _W, _zero_blk, 0)
        if from_x:
            for j in range(_W // 16):
                ones_v[pl.ds(j * 16, 16)] = jnp.ones((16,), jnp.float32)

        for b in range(2):
            _wait_idx()
            pltpu.async_copy(_gather_src(ed_v.at[b, 0]), rows_v.at[b], gsem)
        plsc.subcore_barrier()

        def _step(w, b, issue_next):
            slot = w % 4
            _wait_rows(gsem, b)
            pltpu.async_copy(rows_v.at[b], acc_sh.at[ed_v.at[slot, 1]],
                             ssem, add=True)
            in_rng = (w >= c * _NWC) & (w < (c + 1) * _NWC)
            if from_x:
                @pl.when(in_rng)
                def _():
                    pltpu.async_copy(ones_v, cnt_sh.at[ed_v.at[slot, 1]],
                                     csem, add=True)
            if issue_next:
                _wait_rows(ssem, b)
                if from_x:
                    @pl.when(in_rng)
                    def _():
                        _wait_cnt()
                @pl.when(w + 4 < _NWIN)
                def _():
                    _issue_idx(w + 4, slot)
                _wait_idx()
                pltpu.async_copy(_gather_src(ed_v.at[(w + 2) % 4, 0]),
                                 rows_v.at[b], gsem)

        def _pair(i, carry):
            for b in range(2):
                _step(2 * i + b, b, True)
            return carry

        lax.fori_loop(0, _NWIN // 2 - 1, _pair, 0)
        for b in range(2):
            _step(_NWIN - 2 + b, b, False)
        for b in range(2):
            _wait_rows(ssem, b)
        if from_x:
            @pl.when(c == 1)
            def _():
                _wait_cnt()
                _wait_cnt()

        plsc.subcore_barrier()
        r0 = s * _RPT
        pltpu.sync_copy(acc_sh.at[pl.ds(r0, _RPT)],
                        agg_hbm.at[pl.ds(c * _NPAD + r0, _RPT)])
        if from_x:
            pltpu.sync_copy(cnt_sh.at[pl.ds(r0, _RPT)],
                            cnt_hbm.at[c, pl.ds(r0, _RPT)])

    return pl.kernel(body, out_type=tuple(out_type), mesh=_sc_mesh,
                     scratch_types=scratch)


_sc_agg_x = _make_sc_agg(True)
_sc_agg_h = _make_sc_agg(False)


def _dense_common(agg_lo_ref, agg_hi_ref, x_lo_ref, x_hi_ref, cnt_ref,
                  wl_ref, wr_ref, b_ref):
    cnt = cnt_ref[0] + cnt_ref[1]
    inv = 1.0 / jnp.maximum(cnt, 1.0)
    m = jnp.concatenate([agg_lo_ref[...] * inv, agg_hi_ref[...] * inv], axis=1)
    xx = jnp.concatenate([x_lo_ref[...], x_hi_ref[...]], axis=1)
    h = (jnp.dot(m, wl_ref[...], preferred_element_type=jnp.float32,
                 precision=lax.Precision.HIGHEST)
         + jnp.dot(xx, wr_ref[...], preferred_element_type=jnp.float32,
                   precision=lax.Precision.HIGHEST)
         + b_ref[...])
    return h


def _dense1_body(agg_lo_ref, agg_hi_ref, x_lo_ref, x_hi_ref, cnt_ref,
                 wl_ref, wr_ref, b_ref, h_ref):
    h = _dense_common(agg_lo_ref, agg_hi_ref, x_lo_ref, x_hi_ref, cnt_ref,
                      wl_ref, wr_ref, b_ref)
    h = jnp.maximum(h, 0.0)
    h_ref[0] = h[:, :_DH]
    h_ref[1] = h[:, _DH:]


def _dense2_body(agg_lo_ref, agg_hi_ref, x_lo_ref, x_hi_ref, cnt_ref,
                 wl_ref, wr_ref, b_ref, out_ref):
    h = _dense_common(agg_lo_ref, agg_hi_ref, x_lo_ref, x_hi_ref, cnt_ref,
                      wl_ref, wr_ref, b_ref)
    hmax = jnp.max(h, axis=1, keepdims=True)
    e = jnp.exp(h - hmax)
    lse = jnp.log(jnp.sum(e, axis=1, keepdims=True))
    out_ref[...] = h - hmax - lse


def _cnt_w_b_specs():
    return [
        pl.BlockSpec((_NC, _RB, 1), lambda i: (0, i, 0)),
        pl.BlockSpec((_D, _D), lambda i: (0, 0)),
        pl.BlockSpec((_D, _D), lambda i: (0, 0)),
        pl.BlockSpec((1, _D), lambda i: (0, 0)),
    ]


_dense1 = pl.pallas_call(
    _dense1_body,
    grid=(_NRB,),
    in_specs=[
        pl.BlockSpec((_RB, _DH), lambda i: (i, 0)),
        pl.BlockSpec((_RB, _DH), lambda i: (i + _NRB, 0)),
        pl.BlockSpec((_RB, _DH), lambda i: (i, 0)),
        pl.BlockSpec((_RB, _DH), lambda i: (i, 1)),
    ] + _cnt_w_b_specs(),
    out_specs=pl.BlockSpec((_NC, _RB, _DH), lambda i: (0, i, 0)),
    out_shape=jax.ShapeDtypeStruct((_NC, _NPAD, _DH), jnp.float32),
)

_dense2 = pl.pallas_call(
    _dense2_body,
    grid=(_NRB,),
    in_specs=[
        pl.BlockSpec((_RB, _DH), lambda i: (i, 0)),
        pl.BlockSpec((_RB, _DH), lambda i: (i + _NRB, 0)),
        pl.BlockSpec((_RB, _DH), lambda i: (i, 0)),
        pl.BlockSpec((_RB, _DH), lambda i: (i + _NRB, 0)),
    ] + _cnt_w_b_specs(),
    out_specs=pl.BlockSpec((_RB, _D), lambda i: (i, 0)),
    out_shape=jax.ShapeDtypeStruct((_N, _D), jnp.float32),
)


def kernel(x, edge_index, W1_l, W1_r, b1, W2_l, W2_r, b2):
    ei = edge_index.astype(jnp.int32)
    src, dst = ei[0], ei[1]

    npad_e = _EPAD - _E
    pad_ar = jnp.arange(npad_e, dtype=jnp.int32)
    pad_src = (pad_ar * 577) % _N
    pad_dst = _N + pad_ar % (_NPAD - _N)
    srcp = jnp.concatenate([src, pad_src]).reshape(_NS, _NWIN, _W)
    dstp = jnp.concatenate([dst, pad_dst]).reshape(_NS, _NWIN, _W)
    edpk1 = jnp.stack([jnp.stack([srcp, dstp], axis=2)] * _NC, axis=0)
    edpk2 = jnp.stack([jnp.stack([srcp, dstp], axis=2),
                       jnp.stack([srcp + _NPAD, dstp], axis=2)], axis=0)

    agg1, cnt = _sc_agg_x(edpk1, x)
    cnt2 = cnt.reshape(_NC, _NPAD, 1)
    h3 = _dense1(agg1, agg1, x, x, cnt2, W1_l.T, W1_r.T, b1.reshape(1, _D))
    hcat = h3.reshape(_NC * _NPAD, _DH)
    agg2 = _sc_agg_h(edpk2, hcat)[0]
    return _dense2(agg2, agg2, hcat, hcat, cnt2,
                   W2_l.T, W2_r.T, b2.reshape(1, _D))


# speedup vs baseline: 9.2061x; 1.0269x over previous
import jax
import jax.numpy as jnp
from jax import lax
from jax.experimental import pallas as pl
from jax.experimental.pallas import tpu as pltpu
from jax.experimental.pallas import tpu_sc as plsc

_N = 10000
_D = 256
_DH = 128
_E = 160000
_NC = 2
_NS = 16
_NPAD = 10240
_RPT = _NPAD // _NS
_EPAD = 163840
_EPT = _EPAD // _NS
_W = 128
_NWIN = _EPT // _W
_NWC = _NWIN // _NC
_RB = 1024
_NRB = _NPAD // _RB

_sc_mesh = plsc.VectorSubcoreMesh(core_axis_name="c", subcore_axis_name="s")


def _make_sc_agg(from_x):
    out_type = [jax.ShapeDtypeStruct((_NC * _NPAD, _DH), jnp.float32)]
    if from_x:
        out_type.append(jax.ShapeDtypeStruct((_NC, _NPAD), jnp.float32))

    scratch = (
        pltpu.VMEM((4, 2, _W), jnp.int32),
        pltpu.VMEM((2, _W, _DH), jnp.float32),
        pltpu.VMEM((_W,), jnp.float32),
        pltpu.VMEM_SHARED((_NPAD, _DH), jnp.float32),
        pltpu.VMEM_SHARED((_NPAD,), jnp.float32),
        pltpu.SemaphoreType.DMA,
        pltpu.SemaphoreType.DMA,
        pltpu.SemaphoreType.DMA,
        pltpu.SemaphoreType.DMA,
    )

    def body(edpk_hbm, x_hbm, agg_hbm, *rest):
        if from_x:
            (cnt_hbm, ed_v, rows_v, ones_v, acc_sh, cnt_sh,
             isem, gsem, ssem, csem) = rest
        else:
            (ed_v, rows_v, ones_v, acc_sh, cnt_sh,
             isem, gsem, ssem, csem) = rest
        c = lax.axis_index("c")
        s = lax.axis_index("s")

        def _gather_src(idx_ref):
            if from_x:
                return x_hbm.at[idx_ref, pl.ds(c * _DH, _DH)]
            return x_hbm.at[idx_ref]

        def _dummy_rows_src():
            if from_x:
                return x_hbm.at[pl.ds(0, _W), pl.ds(0, _DH)]
            return x_hbm.at[pl.ds(0, _W)]

        def _dummy_row_src():
            if from_x:
                return x_hbm.at[0, pl.ds(0, _DH)]
            return x_hbm.at[0]

        def _issue_idx(w, slot):
            pltpu.async_copy(edpk_hbm.at[c, s, w], ed_v.at[slot], isem)

        def _wait_cnt():
            pltpu.make_async_copy(_dummy_row_src(), ones_v, csem).wait()

        def _wait_idx():
            pltpu.make_async_copy(edpk_hbm.at[0, 0, 0], ed_v.at[0],
                                  isem).wait()

        def _wait_rows(sem, b):
            pltpu.make_async_copy(_dummy_rows_src(), rows_v.at[b], sem).wait()

        for k in range(4):
            _issue_idx(k, k)

        def _fill_zero_row(i, carry):
            for j in range(_DH // 16):
                rows_v[0, i, pl.ds(j * 16, 16)] = jnp.zeros((16,), jnp.float32)
            return carry

        lax.fori_loop(0, _W, _fill_zero_row, 0)

        def _zero_blk(k, carry):
            r0 = s * _RPT + k * _W
            pltpu.sync_copy(rows_v.at[0], acc_sh.at[pl.ds(r0, _W)])
            if from_x:
                pltpu.sync_copy(rows_v.at[0, 0], cnt_sh.at[pl.ds(r0, _W)])
            return carry

        lax.fori_loop(0, _RPT // _W, _zero_blk, 0)
        if from_x:
            for j in range(_W // 16):
                ones_v[pl.ds(j * 16, 16)] = jnp.ones((16,), jnp.float32)

        for b in range(2):
            _wait_idx()
            pltpu.async_copy(_gather_src(ed_v.at[b, 0]), rows_v.at[b], gsem)
        plsc.subcore_barrier()

        def _step(w, b, issue_next):
            slot = w % 4
            _wait_rows(gsem, b)
            pltpu.async_copy(rows_v.at[b], acc_sh.at[ed_v.at[slot, 1]],
                             ssem, add=True)
            in_rng = (w >= c * _NWC) & (w < (c + 1) * _NWC)
            if from_x:
                @pl.when(in_rng)
                def _():
                    pltpu.async_copy(ones_v, cnt_sh.at[ed_v.at[slot, 1]],
                                     csem, add=True)
            if issue_next:
                _wait_rows(ssem, b)
                if from_x:
                    @pl.when(in_rng)
                    def _():
                        _wait_cnt()
                @pl.when(w + 4 < _NWIN)
                def _():
                    _issue_idx(w + 4, slot)
                _wait_idx()
                pltpu.async_copy(_gather_src(ed_v.at[(w + 2) % 4, 0]),
                                 rows_v.at[b], gsem)

        def _pair(i, carry):
            for b in range(2):
                _step(2 * i + b, b, True)
            return carry

        lax.fori_loop(0, _NWIN // 2 - 1, _pair, 0)
        for b in range(2):
            _step(_NWIN - 2 + b, b, False)
        for b in range(2):
            _wait_rows(ssem, b)
        if from_x:
            @pl.when(c == 1)
            def _():
                _wait_cnt()
                _wait_cnt()

        plsc.subcore_barrier()
        r0 = s * _RPT
        pltpu.sync_copy(acc_sh.at[pl.ds(r0, _RPT)],
                        agg_hbm.at[pl.ds(c * _NPAD + r0, _RPT)])
        if from_x:
            pltpu.sync_copy(cnt_sh.at[pl.ds(r0, _RPT)],
                            cnt_hbm.at[c, pl.ds(r0, _RPT)])

    return pl.kernel(body, out_type=tuple(out_type), mesh=_sc_mesh,
                     scratch_types=scratch)


_sc_agg_x = _make_sc_agg(True)
_sc_agg_h = _make_sc_agg(False)


def _matmul(a, w_ref):
    return jnp.dot(a, w_ref[...], preferred_element_type=jnp.float32,
                   precision=lax.Precision.HIGHEST)


def _mean_wl(agg_lo_ref, agg_hi_ref, cnt_ref, wl_ref, res_ref):
    cnt = cnt_ref[0] + cnt_ref[1]
    inv = 1.0 / jnp.maximum(cnt, 1.0)
    m = jnp.concatenate([agg_lo_ref[...] * inv, agg_hi_ref[...] * inv], axis=1)
    return _matmul(m, wl_ref) + res_ref[...]


def _lin1_body(x_lo_ref, x_hi_ref, wr_ref, b_ref, out_ref):
    xx = jnp.concatenate([x_lo_ref[...], x_hi_ref[...]], axis=1)
    out_ref[...] = _matmul(xx, wr_ref) + b_ref[...]


def _dense1_body(agg_lo_ref, agg_hi_ref, cnt_ref, wl_ref, res_ref, h_ref):
    h = jnp.maximum(_mean_wl(agg_lo_ref, agg_hi_ref, cnt_ref, wl_ref,
                             res_ref), 0.0)
    h_ref[0] = h[:, :_DH]
    h_ref[1] = h[:, _DH:]


def _dense2_body(agg_lo_ref, agg_hi_ref, cnt_ref, wl_ref, res_ref, out_ref):
    h = _mean_wl(agg_lo_ref, agg_hi_ref, cnt_ref, wl_ref, res_ref)
    hmax = jnp.max(h, axis=1, keepdims=True)
    e = jnp.exp(h - hmax)
    lse = jnp.log(jnp.sum(e, axis=1, keepdims=True))
    out_ref[...] = h - hmax - lse


def _lo_spec():
    return pl.BlockSpec((_RB, _DH), lambda i: (i, 0))


def _hi_spec():
    return pl.BlockSpec((_RB, _DH), lambda i: (i + _NRB, 0))


def _w_spec():
    return pl.BlockSpec((_D, _D), lambda i: (0, 0))


def _res_spec():
    return pl.BlockSpec((_RB, _D), lambda i: (i, 0))


def _agg_cnt_wl_specs():
    return [
        _lo_spec(),
        _hi_spec(),
        pl.BlockSpec((_NC, _RB, 1), lambda i: (0, i, 0)),
        _w_spec(),
        _res_spec(),
    ]


_lin1 = pl.pallas_call(
    _lin1_body,
    grid=(_NRB,),
    in_specs=[
        pl.BlockSpec((_RB, _DH), lambda i: (i, 0)),
        pl.BlockSpec((_RB, _DH), lambda i: (i, 1)),
        _w_spec(),
        pl.BlockSpec((1, _D), lambda i: (0, 0)),
    ],
    out_specs=_res_spec(),
    out_shape=jax.ShapeDtypeStruct((_NPAD, _D), jnp.float32),
)

_lin2 = pl.pallas_call(
    _lin1_body,
    grid=(_NRB,),
    in_specs=[
        _lo_spec(),
        _hi_spec(),
        _w_spec(),
        pl.BlockSpec((1, _D), lambda i: (0, 0)),
    ],
    out_specs=_res_spec(),
    out_shape=jax.ShapeDtypeStruct((_NPAD, _D), jnp.float32),
)

_dense1 = pl.pallas_call(
    _dense1_body,
    grid=(_NRB,),
    in_specs=_agg_cnt_wl_specs(),
    out_specs=pl.BlockSpec((_NC, _RB, _DH), lambda i: (0, i, 0)),
    out_shape=jax.ShapeDtypeStruct((_NC, _NPAD, _DH), jnp.float32),
)

_dense2 = pl.pallas_call(
    _dense2_body,
    grid=(_NRB,),
    in_specs=_agg_cnt_wl_specs(),
    out_specs=pl.BlockSpec((_RB, _D), lambda i: (i, 0)),
    out_shape=jax.ShapeDtypeStruct((_N, _D), jnp.float32),
)


def kernel(x, edge_index, W1_l, W1_r, b1, W2_l, W2_r, b2):
    ei = edge_index.astype(jnp.int32)
    src, dst = ei[0], ei[1]

    npad_e = _EPAD - _E
    pad_ar = jnp.arange(npad_e, dtype=jnp.int32)
    pad_src = (pad_ar * 577) % _N
    pad_dst = _N + pad_ar % (_NPAD - _N)
    srcp = jnp.concatenate([src, pad_src]).reshape(_NS, _NWIN, _W)
    dstp = jnp.concatenate([dst, pad_dst]).reshape(_NS, _NWIN, _W)
    edpk1 = jnp.stack([jnp.stack([srcp, dstp], axis=2)] * _NC, axis=0)
    edpk2 = jnp.stack([jnp.stack([srcp, dstp], axis=2),
                       jnp.stack([srcp + _NPAD, dstp], axis=2)], axis=0)

    agg1, cnt = _sc_agg_x(edpk1, x)
    cnt2 = cnt.reshape(_NC, _NPAD, 1)
    xr1 = _lin1(x, x, W1_r.T, b1.reshape(1, _D))
    h3 = _dense1(agg1, agg1, cnt2, W1_l.T, xr1)
    hcat = h3.reshape(_NC * _NPAD, _DH)
    agg2 = _sc_agg_h(edpk2, hcat)
    hr2 = _lin2(hcat, hcat, W2_r.T, b2.reshape(1, _D))
    return _dense2(agg2[0], agg2[0], cnt2, W2_l.T, hr2)
